# baseline (speedup 1.0000x reference)
import functools

import jax
import jax.numpy as jnp
import numpy as np
from jax import lax
from jax.experimental import pallas as pl
from jax.experimental.pallas import tpu as pltpu
from jax.experimental.pallas import tpu_sc as plsc

N = 2048
D = 768
E = 8
H = 4 * D
BLK = 128
NB = 39
NBUF = NB * BLK
NW = 32
TPW = N // NW
LANES = 16
WLANE = 128


def _gating_body(x_ref, gw_ref, gb_ref, vw_ref, vb_ref, nz_ref,
                 q0_ref, q1_ref, w0_ref, w1_ref, steps_ref):
    x = x_ref[...]
    clean = jnp.dot(x, gw_ref[...], preferred_element_type=jnp.float32) + gb_ref[...]
    sv = jnp.dot(x, vw_ref[...], preferred_element_type=jnp.float32) + vb_ref[...]
    sigma = jnp.maximum(sv, 0.0) + jnp.log1p(jnp.exp(-jnp.abs(sv)))
    noisy = clean + nz_ref[...] * sigma

    ioe = lax.broadcasted_iota(jnp.int32, (N, E), 1)
    m1 = jnp.max(noisy, axis=1, keepdims=True)
    i1 = jnp.min(jnp.where(noisy == m1, ioe, E), axis=1, keepdims=True)
    noisy2 = jnp.where(ioe == i1, -jnp.inf, noisy)
    m2 = jnp.max(noisy2, axis=1, keepdims=True)
    i2 = jnp.min(jnp.where(noisy2 == m2, ioe, E), axis=1, keepdims=True)
    d = jnp.exp(m2 - m1)
    w0_ref[...] = jnp.broadcast_to(1.0 / (1.0 + d), (N, WLANE))
    w1_ref[...] = jnp.broadcast_to(d / (1.0 + d), (N, WLANE))

    maskf = ((ioe == i1) | (ioe == i2)).astype(jnp.float32)

    r = lax.broadcasted_iota(jnp.int32, (BLK, BLK), 0)
    c = lax.broadcasted_iota(jnp.int32, (BLK, BLK), 1)
    tril = (c < r).astype(jnp.float32)
    carry = jnp.zeros((1, E), jnp.float32)
    parts = []
    for b in range(N // BLK):
        blk = maskf[b * BLK:(b + 1) * BLK, :]
        parts.append(jnp.dot(tril, blk, preferred_element_type=jnp.float32) + carry)
        carry = carry + jnp.sum(blk, axis=0, keepdims=True)
    csum = jnp.concatenate(parts, axis=0)
    counts = carry.astype(jnp.int32)

    nblk = (counts + (BLK - 1)) // BLK
    nbf = nblk.astype(jnp.float32)
    re_ = lax.broadcasted_iota(jnp.int32, (E, E), 0)
    ce_ = lax.broadcasted_iota(jnp.int32, (E, E), 1)
    triE = (re_ < ce_).astype(jnp.float32)
    bstart = jnp.dot(nbf, triE, preferred_element_type=jnp.float32)
    off = bstart * BLK
    pos = (off + csum).astype(jnp.int32)

    q0_ref[...] = jnp.sum(jnp.where(ioe == i1, pos, 0), axis=1, keepdims=True)
    q1_ref[...] = jnp.sum(jnp.where(ioe == i2, pos, 0), axis=1, keepdims=True)

    bstart_i = bstart.astype(jnp.int32)
    present = (nblk > 0).astype(jnp.int32)
    iob = lax.broadcasted_iota(jnp.int32, (NB, E), 0)
    ioe_b = lax.broadcasted_iota(jnp.int32, (NB, E), 1)
    be = jnp.sum((iob >= bstart_i).astype(jnp.int32), axis=1, keepdims=True) - 1
    sw = jnp.max(present * ((iob == bstart_i) & (nblk > 0)).astype(jnp.int32),
                 axis=1, keepdims=True)
    ordj = jnp.sum(present * (ioe_b < be).astype(jnp.int32), axis=1, keepdims=True)
    pe = lax.rem(ordj, 3)
    fe = jnp.min(jnp.where((present == 1) & (ioe_b > be), ioe_b, E),
                 axis=1, keepdims=True)
    ff = sw * (fe < E).astype(jnp.int32)
    fe2 = jnp.min(jnp.where((present == 1) & (ioe_b > fe), ioe_b, E),
                  axis=1, keepdims=True)
    ff2 = sw * (fe2 < E).astype(jnp.int32)
    fe = jnp.minimum(fe, E - 1)
    fe2 = jnp.minimum(fe2, E - 1)
    t_end = jnp.sum(jnp.where(ioe_b[:1] == E - 1, bstart_i + nblk, 0),
                    axis=1, keepdims=True)
    valid = (iob[:, :1] < t_end).astype(jnp.int32)
    steps_ref[...] = jnp.concatenate([be, sw, pe, fe, ff, valid, fe2, ff2],
                                     axis=1)


def _gating(x, gate_W, gate_b, var_W, var_b, noise):
    return pl.pallas_call(
        _gating_body,
        out_shape=(
            jax.ShapeDtypeStruct((N, 1), jnp.int32),
            jax.ShapeDtypeStruct((N, 1), jnp.int32),
            jax.ShapeDtypeStruct((N, WLANE), jnp.float32),
            jax.ShapeDtypeStruct((N, WLANE), jnp.float32),
            jax.ShapeDtypeStruct((NB, 8), jnp.int32),
        ),
    )(x, gate_W, gate_b.reshape(1, E), var_W, var_b.reshape(1, E), noise)


def _sc_mesh():
    return plsc.VectorSubcoreMesh(core_axis_name="c", subcore_axis_name="s",
                                  num_cores=2, num_subcores=16)


def _dispatch(x, q0, q1, w0, w1):
    @functools.partial(
        pl.kernel,
        out_type=(
            jax.ShapeDtypeStruct((NBUF, D), jnp.float32),
            jax.ShapeDtypeStruct((NBUF, WLANE), jnp.float32),
        ),
        mesh=_sc_mesh(),
        scratch_types=[
            pltpu.VMEM((TPW, D), jnp.float32),
            pltpu.VMEM((TPW, WLANE), jnp.float32),
            pltpu.VMEM((TPW, WLANE), jnp.float32),
            pltpu.VMEM((TPW,), jnp.int32),
            pltpu.VMEM((TPW,), jnp.int32),
            pltpu.SemaphoreType.DMA,
            pltpu.SemaphoreType.DMA,
        ],
    )
    def body(x_hbm, q0_hbm, q1_hbm, w0_hbm, w1_hbm, xs_hbm, ws_hbm,
             rows_v, wrow0_v, wrow1_v, idx0_v, idx1_v, lsem, ssem):
        wid = lax.axis_index("s") * 2 + lax.axis_index("c")
        base = wid * TPW
        sl = pl.ds(base, TPW)
        loads = (pltpu.async_copy(x_hbm.at[sl], rows_v, lsem),
                 pltpu.async_copy(q0_hbm.at[sl], idx0_v, lsem),
                 pltpu.async_copy(q1_hbm.at[sl], idx1_v, lsem),
                 pltpu.async_copy(w0_hbm.at[sl], wrow0_v, lsem),
                 pltpu.async_copy(w1_hbm.at[sl], wrow1_v, lsem))
        for cp in loads:
            cp.wait()
        scat = (pltpu.async_copy(rows_v, xs_hbm.at[idx0_v], ssem),
                pltpu.async_copy(rows_v, xs_hbm.at[idx1_v], ssem),
                pltpu.async_copy(wrow0_v, ws_hbm.at[idx0_v], ssem),
                pltpu.async_copy(wrow1_v, ws_hbm.at[idx1_v], ssem))
        for cp in scat:
            cp.wait()

    return body(x, q0, q1, w0, w1)


def _ffn_body(steps_ref, xs_ref, ws_ref, b1_ref, b2_ref, w1_hbm, w2_hbm,
              ys_ref, w1buf, w2buf, wsem):
    i = pl.program_id(0)
    be_i = steps_ref[i, 0]
    sw_i = steps_ref[i, 1]
    pe_i = steps_ref[i, 2]
    fe_i = steps_ref[i, 3]
    ff_i = steps_ref[i, 4]
    valid_i = steps_ref[i, 5]
    fe2_i = steps_ref[i, 6]
    ff2_i = steps_ref[i, 7]

    def w_fetch(e, p):
        return (pltpu.make_async_copy(w1_hbm.at[e], w1buf.at[p], wsem.at[p]),
                pltpu.make_async_copy(w2_hbm.at[e], w2buf.at[p], wsem.at[p]))

    @pl.when(i == 0)
    def _():
        for cp in w_fetch(be_i, 0):
            cp.start()

    @pl.when((i == 0) & (ff_i == 1))
    def _():
        for cp in w_fetch(fe_i, 1):
            cp.start()

    @pl.when(ff2_i == 1)
    def _():
        for cp in w_fetch(fe2_i, lax.rem(pe_i + 2, 3)):
            cp.start()

    @pl.when(sw_i == 1)
    def _():
        for cp in w_fetch(0, pe_i):
            cp.wait()

    def compute(wp):
        xv = xs_ref[...]
        h = jnp.dot(xv, w1buf[wp], preferred_element_type=jnp.float32)
        h = jnp.maximum(h + b1_ref[pl.ds(be_i, 1)], 0.0)
        y = jnp.dot(h, w2buf[wp], preferred_element_type=jnp.float32)
        ys_ref[...] = (y + b2_ref[pl.ds(be_i, 1)]) * ws_ref[:, :1]

    for wp in range(3):
        @pl.when((pe_i == wp) & (valid_i == 1))
        def _(wp=wp):
            compute(wp)


def _ffn(steps, xs, ws, W1, b1, W2, b2):
    grid_spec = pltpu.PrefetchScalarGridSpec(
        num_scalar_prefetch=1,
        grid=(NB,),
        in_specs=[
            pl.BlockSpec((BLK, D), lambda i, s: (i, 0)),
            pl.BlockSpec((BLK, WLANE), lambda i, s: (i, 0)),
            pl.BlockSpec((E, H), lambda i, s: (0, 0)),
            pl.BlockSpec((E, D), lambda i, s: (0, 0)),
            pl.BlockSpec(memory_space=pl.ANY),
            pl.BlockSpec(memory_space=pl.ANY),
        ],
        out_specs=pl.BlockSpec((BLK, D), lambda i, s: (i, 0)),
        scratch_shapes=[
            pltpu.VMEM((3, D, H), jnp.float32),
            pltpu.VMEM((3, H, D), jnp.float32),
            pltpu.SemaphoreType.DMA((3,)),
        ],
    )
    return pl.pallas_call(
        _ffn_body,
        grid_spec=grid_spec,
        out_shape=jax.ShapeDtypeStruct((NBUF, D), jnp.float32),
    )(steps, xs, ws, b1, b2, W1, W2)


def _combine(ys, q0, q1):
    @functools.partial(
        pl.kernel,
        out_type=jax.ShapeDtypeStruct((N, D), jnp.float32),
        mesh=_sc_mesh(),
        scratch_types=[
            pltpu.VMEM((TPW, D), jnp.float32),
            pltpu.VMEM((TPW, D), jnp.float32),
            pltpu.VMEM((TPW,), jnp.int32),
            pltpu.VMEM((TPW,), jnp.int32),
            pltpu.SemaphoreType.DMA,
        ],
    )
    def body(ys_hbm, q0_hbm, q1_hbm, out_hbm, buf0, buf1, i0, i1, sem):
        wid = lax.axis_index("s") * 2 + lax.axis_index("c")
        base = wid * TPW
        pltpu.sync_copy(q0_hbm.at[pl.ds(base, TPW)], i0)
        pltpu.sync_copy(q1_hbm.at[pl.ds(base, TPW)], i1)
        cp0 = pltpu.async_copy(ys_hbm.at[i0], buf0, sem)
        cp1 = pltpu.async_copy(ys_hbm.at[i1], buf1, sem)
        cp0.wait()
        cp1.wait()

        def per_token(t, _):
            for j in range(D // LANES):
                sl = slice(j * LANES, (j + 1) * LANES)
                buf0[t, sl] = buf0[t, sl] + buf1[t, sl]
            return 0

        lax.fori_loop(0, TPW, per_token, 0)
        pltpu.sync_copy(buf0, out_hbm.at[pl.ds(base, TPW)])

    return body(ys, q0, q1)


_NOISE = np.asarray(jax.random.normal(jax.random.key(1), (N, E), jnp.float32))


def kernel(x, gate_W, gate_b, var_W, var_b, W1, b1, W2, b2):
    noise = jnp.asarray(_NOISE)
    q0, q1, w0, w1, steps = _gating(x, gate_W, gate_b, var_W, var_b, noise)
    q0 = q0.reshape(N)
    q1 = q1.reshape(N)
    xs, ws = _dispatch(x, q0, q1, w0, w1)
    ys = _ffn(steps, xs, ws, W1, b1, W2, b2)
    return _combine(ys, q0, q1)


# speedup vs baseline: 1.0324x; 1.0324x over previous
import functools

import jax
import jax.numpy as jnp
import numpy as np
from jax import lax
from jax.experimental import pallas as pl
from jax.experimental.pallas import tpu as pltpu
from jax.experimental.pallas import tpu_sc as plsc

N = 2048
D = 768
E = 8
H = 4 * D
BLK = 128
NB = 39
NBUF = NB * BLK
NW = 32
TPW = N // NW
LANES = 16
WLANE = 128


def _gating_body(x_ref, gw_ref, gb_ref, vw_ref, vb_ref, nz_ref,
                 q0_ref, q1_ref, w0_ref, w1_ref, steps_ref):
    x = x_ref[...]
    clean = jnp.dot(x, gw_ref[...], preferred_element_type=jnp.float32) + gb_ref[...]
    sv = jnp.dot(x, vw_ref[...], preferred_element_type=jnp.float32) + vb_ref[...]
    sigma = jnp.maximum(sv, 0.0) + jnp.log1p(jnp.exp(-jnp.abs(sv)))
    noisy = clean + nz_ref[...] * sigma

    ioe = lax.broadcasted_iota(jnp.int32, (N, E), 1)
    m1 = jnp.max(noisy, axis=1, keepdims=True)
    i1 = jnp.min(jnp.where(noisy == m1, ioe, E), axis=1, keepdims=True)
    noisy2 = jnp.where(ioe == i1, -jnp.inf, noisy)
    m2 = jnp.max(noisy2, axis=1, keepdims=True)
    i2 = jnp.min(jnp.where(noisy2 == m2, ioe, E), axis=1, keepdims=True)
    d = jnp.exp(m2 - m1)
    w0_ref[...] = jnp.broadcast_to(1.0 / (1.0 + d), (N, WLANE))
    w1_ref[...] = jnp.broadcast_to(d / (1.0 + d), (N, WLANE))

    maskf = ((ioe == i1) | (ioe == i2)).astype(jnp.float32)

    r = lax.broadcasted_iota(jnp.int32, (BLK, BLK), 0)
    c = lax.broadcasted_iota(jnp.int32, (BLK, BLK), 1)
    tril = (c < r).astype(jnp.float32)
    carry = jnp.zeros((1, E), jnp.float32)
    parts = []
    for b in range(N // BLK):
        blk = maskf[b * BLK:(b + 1) * BLK, :]
        parts.append(jnp.dot(tril, blk, preferred_element_type=jnp.float32) + carry)
        carry = carry + jnp.sum(blk, axis=0, keepdims=True)
    csum = jnp.concatenate(parts, axis=0)
    counts = carry.astype(jnp.int32)

    nblk = (counts + (BLK - 1)) // BLK
    nbf = nblk.astype(jnp.float32)
    re_ = lax.broadcasted_iota(jnp.int32, (E, E), 0)
    ce_ = lax.broadcasted_iota(jnp.int32, (E, E), 1)
    triE = (re_ < ce_).astype(jnp.float32)
    bstart = jnp.dot(nbf, triE, preferred_element_type=jnp.float32)
    off = bstart * BLK
    pos = (off + csum).astype(jnp.int32)

    q0_ref[...] = jnp.sum(jnp.where(ioe == i1, pos, 0), axis=1, keepdims=True)
    q1_ref[...] = jnp.sum(jnp.where(ioe == i2, pos, 0), axis=1, keepdims=True)

    bstart_i = bstart.astype(jnp.int32)
    present = (nblk > 0).astype(jnp.int32)
    iob = lax.broadcasted_iota(jnp.int32, (NB, E), 0)
    ioe_b = lax.broadcasted_iota(jnp.int32, (NB, E), 1)
    be = jnp.sum((iob >= bstart_i).astype(jnp.int32), axis=1, keepdims=True) - 1
    sw = jnp.max(present * ((iob == bstart_i) & (nblk > 0)).astype(jnp.int32),
                 axis=1, keepdims=True)
    ordj = jnp.sum(present * (ioe_b < be).astype(jnp.int32), axis=1, keepdims=True)
    pe = lax.rem(ordj, 2)
    fe = jnp.min(jnp.where((present == 1) & (ioe_b > be), ioe_b, E),
                 axis=1, keepdims=True)
    ff = sw * (fe < E).astype(jnp.int32)
    fe = jnp.minimum(fe, E - 1)
    t_end = jnp.sum(jnp.where(ioe_b[:1] == E - 1, bstart_i + nblk, 0),
                    axis=1, keepdims=True)
    valid = (iob[:, :1] < t_end).astype(jnp.int32)
    steps_ref[...] = jnp.concatenate([be, sw, pe, fe, ff, valid], axis=1)


def _gating(x, gate_W, gate_b, var_W, var_b, noise):
    return pl.pallas_call(
        _gating_body,
        out_shape=(
            jax.ShapeDtypeStruct((N, 1), jnp.int32),
            jax.ShapeDtypeStruct((N, 1), jnp.int32),
            jax.ShapeDtypeStruct((N, WLANE), jnp.float32),
            jax.ShapeDtypeStruct((N, WLANE), jnp.float32),
            jax.ShapeDtypeStruct((NB, 6), jnp.int32),
        ),
    )(x, gate_W, gate_b.reshape(1, E), var_W, var_b.reshape(1, E), noise)


def _sc_mesh():
    return plsc.VectorSubcoreMesh(core_axis_name="c", subcore_axis_name="s",
                                  num_cores=2, num_subcores=16)


def _dispatch(x, q0, q1, w0, w1):
    @functools.partial(
        pl.kernel,
        out_type=(
            jax.ShapeDtypeStruct((NBUF, D), jnp.float32),
            jax.ShapeDtypeStruct((NBUF, WLANE), jnp.float32),
        ),
        mesh=_sc_mesh(),
        scratch_types=[
            pltpu.VMEM((TPW, D), jnp.float32),
            pltpu.VMEM((TPW, WLANE), jnp.float32),
            pltpu.VMEM((TPW, WLANE), jnp.float32),
            pltpu.VMEM((TPW,), jnp.int32),
            pltpu.VMEM((TPW,), jnp.int32),
            pltpu.SemaphoreType.DMA,
            pltpu.SemaphoreType.DMA,
        ],
    )
    def body(x_hbm, q0_hbm, q1_hbm, w0_hbm, w1_hbm, xs_hbm, ws_hbm,
             rows_v, wrow0_v, wrow1_v, idx0_v, idx1_v, lsem, ssem):
        wid = lax.axis_index("s") * 2 + lax.axis_index("c")
        base = wid * TPW
        sl = pl.ds(base, TPW)
        loads = (pltpu.async_copy(x_hbm.at[sl], rows_v, lsem),
                 pltpu.async_copy(q0_hbm.at[sl], idx0_v, lsem),
                 pltpu.async_copy(q1_hbm.at[sl], idx1_v, lsem),
                 pltpu.async_copy(w0_hbm.at[sl], wrow0_v, lsem),
                 pltpu.async_copy(w1_hbm.at[sl], wrow1_v, lsem))
        for cp in loads:
            cp.wait()
        scat = (pltpu.async_copy(rows_v, xs_hbm.at[idx0_v], ssem),
                pltpu.async_copy(rows_v, xs_hbm.at[idx1_v], ssem),
                pltpu.async_copy(wrow0_v, ws_hbm.at[idx0_v], ssem),
                pltpu.async_copy(wrow1_v, ws_hbm.at[idx1_v], ssem))
        for cp in scat:
            cp.wait()

    return body(x, q0, q1, w0, w1)


def _ffn_body(steps_ref, xs_ref, ws_ref, b1_ref, b2_ref, w1_hbm, w2_hbm,
              ys_ref, w1buf, w2buf, wsem):
    i = pl.program_id(0)
    be_i = steps_ref[i, 0]
    sw_i = steps_ref[i, 1]
    pe_i = steps_ref[i, 2]
    fe_i = steps_ref[i, 3]
    ff_i = steps_ref[i, 4]
    valid_i = steps_ref[i, 5]

    def w_fetch(e, p):
        return (pltpu.make_async_copy(w1_hbm.at[e], w1buf.at[p], wsem.at[p]),
                pltpu.make_async_copy(w2_hbm.at[e], w2buf.at[p], wsem.at[p]))

    @pl.when(i == 0)
    def _():
        for cp in w_fetch(be_i, 0):
            cp.start()

    @pl.when(ff_i == 1)
    def _():
        for cp in w_fetch(fe_i, 1 - pe_i):
            cp.start()

    @pl.when(sw_i == 1)
    def _():
        for cp in w_fetch(0, pe_i):
            cp.wait()

    def compute(wp):
        xv = xs_ref[...]
        h = jnp.dot(xv, w1buf[wp], preferred_element_type=jnp.float32)
        h = jnp.maximum(h + b1_ref[pl.ds(be_i, 1)], 0.0)
        y = jnp.dot(h, w2buf[wp], preferred_element_type=jnp.float32)
        ys_ref[...] = (y + b2_ref[pl.ds(be_i, 1)]) * ws_ref[:, :1]

    @pl.when((pe_i == 0) & (valid_i == 1))
    def _():
        compute(0)

    @pl.when((pe_i == 1) & (valid_i == 1))
    def _():
        compute(1)


def _ffn(steps, xs, ws, W1, b1, W2, b2):
    grid_spec = pltpu.PrefetchScalarGridSpec(
        num_scalar_prefetch=1,
        grid=(NB,),
        in_specs=[
            pl.BlockSpec((BLK, D), lambda i, s: (i, 0)),
            pl.BlockSpec((BLK, WLANE), lambda i, s: (i, 0)),
            pl.BlockSpec((E, H), lambda i, s: (0, 0)),
            pl.BlockSpec((E, D), lambda i, s: (0, 0)),
            pl.BlockSpec(memory_space=pl.ANY),
            pl.BlockSpec(memory_space=pl.ANY),
        ],
        out_specs=pl.BlockSpec((BLK, D), lambda i, s: (i, 0)),
        scratch_shapes=[
            pltpu.VMEM((2, D, H), jnp.float32),
            pltpu.VMEM((2, H, D), jnp.float32),
            pltpu.SemaphoreType.DMA((2,)),
        ],
    )
    return pl.pallas_call(
        _ffn_body,
        grid_spec=grid_spec,
        out_shape=jax.ShapeDtypeStruct((NBUF, D), jnp.float32),
    )(steps, xs, ws, b1, b2, W1, W2)


def _combine(ys, q0, q1):
    @functools.partial(
        pl.kernel,
        out_type=jax.ShapeDtypeStruct((N, D), jnp.float32),
        mesh=_sc_mesh(),
        scratch_types=[
            pltpu.VMEM((TPW, D), jnp.float32),
            pltpu.VMEM((TPW, D), jnp.float32),
            pltpu.VMEM((TPW,), jnp.int32),
            pltpu.VMEM((TPW,), jnp.int32),
            pltpu.SemaphoreType.DMA,
            pltpu.SemaphoreType.DMA,
        ],
    )
    def body(ys_hbm, q0_hbm, q1_hbm, out_hbm, buf0, buf1, i0, i1, sem, osem):
        wid = lax.axis_index("s") * 2 + lax.axis_index("c")
        base = wid * TPW
        half = TPW // 2
        pltpu.sync_copy(q0_hbm.at[pl.ds(base, TPW)], i0)
        pltpu.sync_copy(q1_hbm.at[pl.ds(base, TPW)], i1)

        def gathers(part):
            hs = pl.ds(part * half, half)
            return (pltpu.async_copy(ys_hbm.at[i0.at[hs]], buf0.at[hs], sem),
                    pltpu.async_copy(ys_hbm.at[i1.at[hs]], buf1.at[hs], sem))

        g0 = gathers(0)
        g1 = gathers(1)

        def add_half(part):
            def per_token(t, _):
                for j in range(D // LANES):
                    sl = slice(j * LANES, (j + 1) * LANES)
                    buf0[t, sl] = buf0[t, sl] + buf1[t, sl]
                return 0
            lax.fori_loop(part * half, (part + 1) * half, per_token, 0)

        for cp in g0:
            cp.wait()
        add_half(0)
        st0 = pltpu.async_copy(buf0.at[pl.ds(0, half)],
                               out_hbm.at[pl.ds(base, half)], osem)
        for cp in g1:
            cp.wait()
        add_half(1)
        st0.wait()
        pltpu.sync_copy(buf0.at[pl.ds(half, half)],
                        out_hbm.at[pl.ds(base + half, half)])

    return body(ys, q0, q1)


_NOISE = np.asarray(jax.random.normal(jax.random.key(1), (N, E), jnp.float32))


def kernel(x, gate_W, gate_b, var_W, var_b, W1, b1, W2, b2):
    noise = jnp.asarray(_NOISE)
    q0, q1, w0, w1, steps = _gating(x, gate_W, gate_b, var_W, var_b, noise)
    q0 = q0.reshape(N)
    q1 = q1.reshape(N)
    xs, ws = _dispatch(x, q0, q1, w0, w1)
    ys = _ffn(steps, xs, ws, W1, b1, W2, b2)
    return _combine(ys, q0, q1)
